# Initial kernel scaffold; baseline (speedup 1.0000x reference)
import jax
import jax.numpy as jnp
from jax.experimental import pallas as pl


def kernel(query, reference_points, input_flatten, input_spatial_shapes, input_level_start_index, W_value, b_value, W_offsets, b_offsets, W_attn, b_attn, W_out, b_out):
    raise NotImplementedError("write your pallas kernel here")


# speedup vs baseline: 6.4137x; 6.4137x over previous
import functools

import numpy as np
import jax
import jax.numpy as jnp
from jax import lax
from jax.experimental import pallas as pl
from jax.experimental.pallas import tpu as pltpu
from jax.experimental.pallas import tpu_sc as plsc

_D_MODEL = 256
_N_HEADS = 8
_N_LEVELS = 4
_N_POINTS = 4
_DH = _D_MODEL // _N_HEADS
_SPATIAL = np.array([[16, 64, 64], [16, 32, 32], [16, 16, 16], [8, 8, 8]], dtype=np.int64)
_LEVEL_START = np.array([0, 65536, 81920, 86016], dtype=np.int64)
_LEN_IN = int(_SPATIAL.prod(axis=1).sum())
_N = 2
_LQ = 900
_NQ = _N * _LQ
_ROWS = _NQ * _N_HEADS
_K = _N_LEVELS * _N_POINTS * 8

_COL = np.arange(_N_HEADS * _N_LEVELS * _N_POINTS)
_L_OF = (_COL % 16) // 4
_H_OF = _COL // 16
_CONST_I = np.stack([
    _SPATIAL[:, 2][_L_OF],
    _SPATIAL[:, 1][_L_OF],
    _LEVEL_START[_L_OF],
    _H_OF,
    _L_OF,
]).astype(np.int32)
_CONST_F = np.stack([
    _SPATIAL[:, 2][_L_OF],
    _SPATIAL[:, 1][_L_OF],
    _SPATIAL[:, 0][_L_OF],
]).astype(np.float32)
_BDIAG = (( _COL[:, None] // 16) == (_COL[None, :] // 16)).astype(np.float32)

_MM_BM = 512
_PREP_BM = 360

_SC_NC = 2
_SC_NS = 16
_NW = _SC_NC * _SC_NS
_SC_B = 8
_RPW = 456
_ROWS_PAD = _NW * _RPW
_SC_T = _RPW // _SC_B
_SC_S = 8
_SC_G = _SC_B * _K // _SC_S


def _matmul_bias(x, w, b, bm):
    m, k = x.shape
    nc = w.shape[1]

    def body(x_ref, w_ref, b_ref, o_ref):
        o_ref[...] = (
            jnp.dot(x_ref[...], w_ref[...], preferred_element_type=jnp.float32)
            + b_ref[...]
        )

    return pl.pallas_call(
        body,
        grid=(m // bm,),
        in_specs=[
            pl.BlockSpec((bm, k), lambda i: (i, 0)),
            pl.BlockSpec((k, nc), lambda i: (0, 0)),
            pl.BlockSpec((1, nc), lambda i: (0, 0)),
        ],
        out_specs=pl.BlockSpec((bm, nc), lambda i: (i, 0)),
        out_shape=jax.ShapeDtypeStruct((m, nc), jnp.float32),
        interpret=False,
    )(x, w, b.reshape(1, nc))


def _prep(q2, rp2, w_off_xyz, b_off_xyz, w_attn, b_attn):
    def body(q_ref, rp_ref, woff_ref, boff_ref, wattn_ref, battn_ref,
             ci_ref, cf_ref, bd_ref, oidx_ref, ow_ref):
        q = q_ref[...]
        bm = q.shape[0]
        offs = []
        for cdim in range(3):
            offs.append(
                jnp.dot(q, woff_ref[:, cdim * 128:(cdim + 1) * 128],
                        preferred_element_type=jnp.float32)
                + boff_ref[:, cdim * 128:(cdim + 1) * 128]
            )
        logits = (jnp.dot(q, wattn_ref[...], preferred_element_type=jnp.float32)
                  + battn_ref[...])
        mx = jnp.max(logits, axis=1, keepdims=True)
        e = jnp.exp(logits - mx)
        s = jnp.dot(e, bd_ref[...], preferred_element_type=jnp.float32)
        attn = e / s

        lcol = ci_ref[4:5, :]
        wf = cf_ref[0:1, :]
        hf = cf_ref[1:2, :]
        df = cf_ref[2:3, :]
        wi = ci_ref[0:1, :]
        hi = ci_ref[1:2, :]
        starti = ci_ref[2:3, :]
        headi = ci_ref[3:4, :]

        rp = rp_ref[...]

        def pick(base):
            return jnp.where(
                lcol == 0, rp[:, base + 0:base + 1],
                jnp.where(lcol == 1, rp[:, base + 3:base + 4],
                          jnp.where(lcol == 2, rp[:, base + 6:base + 7],
                                    rp[:, base + 9:base + 10])))

        x = pick(0) * wf - 0.5 + offs[0]
        y = pick(1) * hf - 0.5 + offs[1]
        z = pick(2) * df - 0.5 + offs[2]

        x0 = jnp.floor(x)
        y0 = jnp.floor(y)
        z0 = jnp.floor(z)
        fx = x - x0
        fy = y - y0
        fz = z - z0

        row0 = pl.program_id(0) * bm
        rid = row0 + lax.broadcasted_iota(jnp.int32, (bm, 1), 0)
        n_off = jnp.where(rid >= _LQ, np.int32(_LEN_IN), np.int32(0))

        for c in range(8):
            dz, dy, dx = (c >> 2) & 1, (c >> 1) & 1, c & 1
            xi = x0 + dx
            yi = y0 + dy
            zi = z0 + dz
            wx = fx if dx else 1.0 - fx
            wy = fy if dy else 1.0 - fy
            wz = fz if dz else 1.0 - fz
            valid = ((xi >= 0) & (xi < wf) & (yi >= 0) & (yi < hf)
                     & (zi >= 0) & (zi < df))
            xi_c = jnp.clip(xi, 0.0, wf - 1.0).astype(jnp.int32)
            yi_c = jnp.clip(yi, 0.0, hf - 1.0).astype(jnp.int32)
            zi_c = jnp.clip(zi, 0.0, df - 1.0).astype(jnp.int32)
            flat = (zi_c * hi + yi_c) * wi + xi_c
            row = (n_off + starti + flat) * _N_HEADS + headi
            wgt = wx * wy * wz * valid.astype(jnp.float32) * attn
            oidx_ref[:, c * 128:(c + 1) * 128] = row
            ow_ref[:, c * 128:(c + 1) * 128] = wgt

    nq = q2.shape[0]
    bm = _PREP_BM
    full = lambda shape: pl.BlockSpec(shape, lambda i: tuple(0 for _ in shape))
    return pl.pallas_call(
        body,
        grid=(nq // bm,),
        in_specs=[
            pl.BlockSpec((bm, _D_MODEL), lambda i: (i, 0)),
            pl.BlockSpec((bm, 12), lambda i: (i, 0)),
            full((_D_MODEL, 384)),
            full((1, 384)),
            full((_D_MODEL, 128)),
            full((1, 128)),
            full((5, 128)),
            full((3, 128)),
            full((128, 128)),
        ],
        out_specs=[
            pl.BlockSpec((bm, 1024), lambda i: (i, 0)),
            pl.BlockSpec((bm, 1024), lambda i: (i, 0)),
        ],
        out_shape=[
            jax.ShapeDtypeStruct((nq, 1024), jnp.int32),
            jax.ShapeDtypeStruct((nq, 1024), jnp.float32),
        ],
        interpret=False,
    )(q2, rp2, w_off_xyz, b_off_xyz, w_attn, b_attn,
      jnp.asarray(_CONST_I), jnp.asarray(_CONST_F), jnp.asarray(_BDIAG))


def _sc_combine(value_rows, idx, w):
    mesh = plsc.VectorSubcoreMesh(core_axis_name="c", subcore_axis_name="s")

    @functools.partial(
        pl.kernel,
        mesh=mesh,
        compiler_params=pltpu.CompilerParams(use_tc_tiling_on_sc=False),
        out_type=jax.ShapeDtypeStruct((_ROWS_PAD, _DH), jnp.float32),
        scratch_types=[
            pltpu.VMEM((_SC_B, _K), jnp.int32),
            pltpu.VMEM((_SC_B, _K), jnp.float32),
            pltpu.VMEM((_SC_B, _K, _DH), jnp.float32),
            pltpu.VMEM((_SC_B, _DH), jnp.float32),
            pltpu.SemaphoreType.DMA,
        ],
    )
    def sc_kernel(value_hbm, idx_hbm, w_hbm, out_hbm,
                  idx_v, w_v, rows_v, acc_v, sem):
        wid = lax.axis_index("s") * _SC_NC + lax.axis_index("c")
        base0 = wid * _RPW

        def chunk(t, carry):
            base = base0 + t * _SC_B
            pltpu.sync_copy(idx_hbm.at[pl.ds(base, _SC_B)], idx_v)
            pltpu.sync_copy(w_hbm.at[pl.ds(base, _SC_B)], w_v)
            for j in range(_SC_B):
                pltpu.async_copy(value_hbm.at[idx_v.at[j]],
                                 rows_v.at[j], sem)
            for j in range(_SC_B):
                pltpu.make_async_copy(value_hbm.at[idx_v.at[j]],
                                      rows_v.at[j], sem).wait()

            def jbody(j, carry2):
                rv = rows_v.at[j]
                wr = w_v.at[j]
                a0 = jnp.zeros((16,), jnp.float32)
                a1 = jnp.zeros((16,), jnp.float32)
                for k16 in range(_K // 16):
                    wv = wr[pl.ds(k16 * 16, 16)]
                    for i in range(16):
                        wk = wv[i]
                        k = k16 * 16 + i
                        a0 = a0 + wk * rv[k, pl.ds(0, 16)]
                        a1 = a1 + wk * rv[k, pl.ds(16, 16)]
                acc_v[j, pl.ds(0, 16)] = a0
                acc_v[j, pl.ds(16, 16)] = a1
                return carry2

            lax.fori_loop(0, _SC_B, jbody, 0)
            pltpu.sync_copy(acc_v, out_hbm.at[pl.ds(base, _SC_B)])
            return carry

        lax.fori_loop(0, _SC_T, chunk, 0)

    return sc_kernel(value_rows, idx, w)


def kernel(query, reference_points, input_flatten, input_spatial_shapes,
           input_level_start_index, W_value, b_value, W_offsets, b_offsets,
           W_attn, b_attn, W_out, b_out):
    n, lq, c = query.shape

    xin = input_flatten.reshape(n * _LEN_IN, c)
    value = _matmul_bias(xin, W_value, b_value, _MM_BM)
    value_rows = value.reshape(n * _LEN_IN * _N_HEADS, _DH)

    w_off_xyz = jnp.transpose(
        W_offsets.reshape(c, 128, 3), (0, 2, 1)).reshape(c, 384)
    b_off_xyz = jnp.transpose(
        b_offsets.reshape(128, 3), (1, 0)).reshape(1, 384)
    q2 = query.reshape(_NQ, c)
    rp2 = reference_points.reshape(_NQ, _N_LEVELS * 3)
    idx2, w2 = _prep(q2, rp2, w_off_xyz, b_off_xyz, W_attn,
                     b_attn.reshape(1, 128))

    pad = _ROWS_PAD - _ROWS
    idx = idx2.reshape(_NQ, 8, _N_HEADS, 16).transpose(0, 2, 3, 1)
    idx = jnp.pad(idx.reshape(_ROWS, _K), ((0, pad), (0, 0)))
    wts = w2.reshape(_NQ, 8, _N_HEADS, 16).transpose(0, 2, 3, 1)
    wts = jnp.pad(wts.reshape(_ROWS, _K), ((0, pad), (0, 0)))

    attn_out = _sc_combine(value_rows, idx, wts)[:_ROWS]

    o2 = _matmul_bias(attn_out.reshape(_NQ, c), W_out, b_out, _PREP_BM)
    return o2.reshape(n, lq, c)


# speedup vs baseline: 7.3353x; 1.1437x over previous
import functools

import numpy as np
import jax
import jax.numpy as jnp
from jax import lax
from jax.experimental import pallas as pl
from jax.experimental.pallas import tpu as pltpu
from jax.experimental.pallas import tpu_sc as plsc

_D_MODEL = 256
_N_HEADS = 8
_N_LEVELS = 4
_N_POINTS = 4
_DH = _D_MODEL // _N_HEADS
_SPATIAL = np.array([[16, 64, 64], [16, 32, 32], [16, 16, 16], [8, 8, 8]], dtype=np.int64)
_LEVEL_START = np.array([0, 65536, 81920, 86016], dtype=np.int64)
_LEN_IN = int(_SPATIAL.prod(axis=1).sum())
_N = 2
_LQ = 900
_NQ = _N * _LQ
_ROWS = _NQ * _N_HEADS
_K = _N_LEVELS * _N_POINTS * 8

_COL = np.arange(_N_HEADS * _N_LEVELS * _N_POINTS)
_L_OF = (_COL % 16) // 4
_H_OF = _COL // 16
_CONST_I = np.stack([
    _SPATIAL[:, 2][_L_OF],
    _SPATIAL[:, 1][_L_OF],
    _LEVEL_START[_L_OF],
    _H_OF,
    _L_OF,
]).astype(np.int32)
_CONST_F = np.stack([
    _SPATIAL[:, 2][_L_OF],
    _SPATIAL[:, 1][_L_OF],
    _SPATIAL[:, 0][_L_OF],
]).astype(np.float32)
_BDIAG = (( _COL[:, None] // 16) == (_COL[None, :] // 16)).astype(np.float32)

_MM_BM = 512
_PREP_BM = 360

_SC_NC = 2
_SC_NS = 16
_NW = _SC_NC * _SC_NS
_SC_B = 8
_RPW = 456
_ROWS_PAD = _NW * _RPW
_SC_T = _RPW // _SC_B
_SC_S = 8
_SC_G = _SC_B * _K // _SC_S


def _matmul_bias(x, w, b, bm):
    m, k = x.shape
    nc = w.shape[1]

    def body(x_ref, w_ref, b_ref, o_ref):
        o_ref[...] = (
            jnp.dot(x_ref[...], w_ref[...], preferred_element_type=jnp.float32)
            + b_ref[...]
        )

    return pl.pallas_call(
        body,
        grid=(m // bm,),
        in_specs=[
            pl.BlockSpec((bm, k), lambda i: (i, 0)),
            pl.BlockSpec((k, nc), lambda i: (0, 0)),
            pl.BlockSpec((1, nc), lambda i: (0, 0)),
        ],
        out_specs=pl.BlockSpec((bm, nc), lambda i: (i, 0)),
        out_shape=jax.ShapeDtypeStruct((m, nc), jnp.float32),
        interpret=False,
    )(x, w, b.reshape(1, nc))


def _prep(q2, rp2, w_off_xyz, b_off_xyz, w_attn, b_attn):
    def body(q_ref, rp_ref, woff_ref, boff_ref, wattn_ref, battn_ref,
             ci_ref, cf_ref, bd_ref, oidx_ref, ow_ref):
        q = q_ref[...]
        bm = q.shape[0]
        offs = []
        for cdim in range(3):
            offs.append(
                jnp.dot(q, woff_ref[:, cdim * 128:(cdim + 1) * 128],
                        preferred_element_type=jnp.float32)
                + boff_ref[:, cdim * 128:(cdim + 1) * 128]
            )
        logits = (jnp.dot(q, wattn_ref[...], preferred_element_type=jnp.float32)
                  + battn_ref[...])
        mx = jnp.max(logits, axis=1, keepdims=True)
        e = jnp.exp(logits - mx)
        s = jnp.dot(e, bd_ref[...], preferred_element_type=jnp.float32)
        attn = e / s

        lcol = ci_ref[4:5, :]
        wf = cf_ref[0:1, :]
        hf = cf_ref[1:2, :]
        df = cf_ref[2:3, :]
        wi = ci_ref[0:1, :]
        hi = ci_ref[1:2, :]
        starti = ci_ref[2:3, :]
        headi = ci_ref[3:4, :]

        rp = rp_ref[...]

        def pick(base):
            return jnp.where(
                lcol == 0, rp[:, base + 0:base + 1],
                jnp.where(lcol == 1, rp[:, base + 3:base + 4],
                          jnp.where(lcol == 2, rp[:, base + 6:base + 7],
                                    rp[:, base + 9:base + 10])))

        x = pick(0) * wf - 0.5 + offs[0]
        y = pick(1) * hf - 0.5 + offs[1]
        z = pick(2) * df - 0.5 + offs[2]

        x0 = jnp.floor(x)
        y0 = jnp.floor(y)
        z0 = jnp.floor(z)
        fx = x - x0
        fy = y - y0
        fz = z - z0

        row0 = pl.program_id(0) * bm
        rid = row0 + lax.broadcasted_iota(jnp.int32, (bm, 1), 0)
        n_off = jnp.where(rid >= _LQ, np.int32(_LEN_IN), np.int32(0))

        for c in range(8):
            dz, dy, dx = (c >> 2) & 1, (c >> 1) & 1, c & 1
            xi = x0 + dx
            yi = y0 + dy
            zi = z0 + dz
            wx = fx if dx else 1.0 - fx
            wy = fy if dy else 1.0 - fy
            wz = fz if dz else 1.0 - fz
            valid = ((xi >= 0) & (xi < wf) & (yi >= 0) & (yi < hf)
                     & (zi >= 0) & (zi < df))
            xi_c = jnp.clip(xi, 0.0, wf - 1.0).astype(jnp.int32)
            yi_c = jnp.clip(yi, 0.0, hf - 1.0).astype(jnp.int32)
            zi_c = jnp.clip(zi, 0.0, df - 1.0).astype(jnp.int32)
            flat = (zi_c * hi + yi_c) * wi + xi_c
            row = (n_off + starti + flat) * _N_HEADS + headi
            wgt = wx * wy * wz * valid.astype(jnp.float32) * attn
            oidx_ref[:, c * 128:(c + 1) * 128] = row
            ow_ref[:, c * 128:(c + 1) * 128] = wgt

    nq = q2.shape[0]
    bm = _PREP_BM
    full = lambda shape: pl.BlockSpec(shape, lambda i: tuple(0 for _ in shape))
    return pl.pallas_call(
        body,
        grid=(nq // bm,),
        in_specs=[
            pl.BlockSpec((bm, _D_MODEL), lambda i: (i, 0)),
            pl.BlockSpec((bm, 12), lambda i: (i, 0)),
            full((_D_MODEL, 384)),
            full((1, 384)),
            full((_D_MODEL, 128)),
            full((1, 128)),
            full((5, 128)),
            full((3, 128)),
            full((128, 128)),
        ],
        out_specs=[
            pl.BlockSpec((bm, 1024), lambda i: (i, 0)),
            pl.BlockSpec((bm, 1024), lambda i: (i, 0)),
        ],
        out_shape=[
            jax.ShapeDtypeStruct((nq, 1024), jnp.int32),
            jax.ShapeDtypeStruct((nq, 1024), jnp.float32),
        ],
        interpret=False,
    )(q2, rp2, w_off_xyz, b_off_xyz, w_attn, b_attn,
      jnp.asarray(_CONST_I), jnp.asarray(_CONST_F), jnp.asarray(_BDIAG))


def _sc_combine(value_rows, idx, w):
    mesh = plsc.VectorSubcoreMesh(core_axis_name="c", subcore_axis_name="s")

    @functools.partial(
        pl.kernel,
        mesh=mesh,
        compiler_params=pltpu.CompilerParams(use_tc_tiling_on_sc=False),
        out_type=jax.ShapeDtypeStruct((_ROWS_PAD, _DH), jnp.float32),
        scratch_types=[
            pltpu.VMEM((_SC_B, _K), jnp.int32),
            pltpu.VMEM((_SC_B, _K), jnp.float32),
            pltpu.VMEM((_SC_B, _K, _DH), jnp.float32),
            pltpu.VMEM((_SC_B, _DH), jnp.float32),
            pltpu.SemaphoreType.DMA,
        ],
    )
    def sc_kernel(value_hbm, idx_hbm, w_hbm, out_hbm,
                  idx_v, w_v, rows_v, acc_v, sem):
        wid = lax.axis_index("s") * _SC_NC + lax.axis_index("c")
        base0 = wid * _RPW

        def chunk(t, carry):
            base = base0 + t * _SC_B
            pltpu.sync_copy(idx_hbm.at[pl.ds(base, _SC_B)], idx_v)
            pltpu.sync_copy(w_hbm.at[pl.ds(base, _SC_B)], w_v)
            for j in range(_SC_B):
                pltpu.async_copy(value_hbm.at[idx_v.at[j]],
                                 rows_v.at[j], sem)
            for j in range(_SC_B):
                pltpu.make_async_copy(value_hbm.at[idx_v.at[j]],
                                      rows_v.at[j], sem).wait()

            def jbody(j, carry2):
                rv = rows_v.at[j]
                wr = w_v.at[j]
                a0 = jnp.zeros((16,), jnp.float32)
                a1 = jnp.zeros((16,), jnp.float32)
                for k16 in range(_K // 16):
                    wv = wr[pl.ds(k16 * 16, 16)]
                    for i in range(16):
                        wk = wv[i]
                        k = k16 * 16 + i
                        a0 = a0 + wk * rv[k, pl.ds(0, 16)]
                        a1 = a1 + wk * rv[k, pl.ds(16, 16)]
                acc_v[j, pl.ds(0, 16)] = a0
                acc_v[j, pl.ds(16, 16)] = a1
                return carry2

            lax.fori_loop(0, _SC_B, jbody, 0)
            pltpu.sync_copy(acc_v, out_hbm.at[pl.ds(base, _SC_B)])
            return carry

        lax.fori_loop(0, _SC_T, chunk, 0)

    return sc_kernel(value_rows, idx, w)


def kernel(query, reference_points, input_flatten, input_spatial_shapes,
           input_level_start_index, W_value, b_value, W_offsets, b_offsets,
           W_attn, b_attn, W_out, b_out):
    n, lq, c = query.shape

    xin = input_flatten.reshape(n * _LEN_IN, c)
    value = _matmul_bias(xin, W_value, b_value, _MM_BM)
    value_rows = value.reshape(n * _LEN_IN * _N_HEADS, _DH)

    w_off_xyz = jnp.transpose(
        W_offsets.reshape(c, 128, 3), (0, 2, 1)).reshape(c, 384)
    b_off_xyz = jnp.transpose(
        b_offsets.reshape(128, 3), (1, 0)).reshape(1, 384)
    q2 = query.reshape(_NQ, c)
    rp2 = reference_points.reshape(_NQ, _N_LEVELS * 3)
    idx2, w2 = _prep(q2, rp2, w_off_xyz, b_off_xyz, W_attn,
                     b_attn.reshape(1, 128))

    pad = _ROWS_PAD - _ROWS
    idx = idx2.reshape(_NQ, 8, _N_HEADS, 16).transpose(0, 2, 1, 3)
    idx = jnp.pad(idx.reshape(_ROWS, _K), ((0, pad), (0, 0)))
    wts = w2.reshape(_NQ, 8, _N_HEADS, 16).transpose(0, 2, 1, 3)
    wts = jnp.pad(wts.reshape(_ROWS, _K), ((0, pad), (0, 0)))

    attn_out = _sc_combine(value_rows, idx, wts)[:_ROWS]

    o2 = _matmul_bias(attn_out.reshape(_NQ, c), W_out, b_out, _PREP_BM)
    return o2.reshape(n, lq, c)
